# Initial kernel scaffold; baseline (speedup 1.0000x reference)
import jax
import jax.numpy as jnp
from jax.experimental import pallas as pl


def kernel(heads, relations, tails, years, months, days, intervals, edge_index, edge_type, ent_r, ent_i, rel_r, rel_i, y_amp, m_amp, d_amp, y_freq, m_freq, d_freq, y_phi, m_phi, d_phi, W_in1, W_out1, W_loop1, w_rel1, b1, loop_r1, loop_i1, W_in2, W_out2, W_loop2, w_rel2, b2, loop_r2, loop_i2):
    raise NotImplementedError("write your pallas kernel here")


# speedup vs baseline: 4.2712x; 4.2712x over previous
import functools

import jax
import jax.numpy as jnp
from jax import lax
from jax.experimental import pallas as pl
from jax.experimental.pallas import tpu as pltpu
from jax.experimental.pallas import tpu_sc as plsc

NE = 10000
NR2 = 400
D = 128
DH = 64
DT = 64
NT = NE + NR2
TW = 9 * DT + DT
E = 320000
HALF = E // 2
B = 1024

NC = 2
NS = 16
NW = NC * NS
LANES = 16

C = 128
NCHUNK = HALF // C
ROW_STRIDE = 624
ROW_SPAN = 640
ZROWS = 160

EPT = E // NW
DEG_C = 2000


@functools.lru_cache(maxsize=None)
def _mesh():
    return plsc.VectorSubcoreMesh(core_axis_name="c", subcore_axis_name="s",
                                  num_cores=NC, num_subcores=NS)


def _sc_agg_body(src_hbm, dst_hbm, et_hbm, x0, x1, r0t, r1t, zeros_hbm,
                 out_in, out_out,
                 acc, src_v, dst_v, et_v, xb, rb):
    c = lax.axis_index("c")
    s = lax.axis_index("s")
    row0 = s * ROW_STRIDE

    def run_half(half_base, o_ref):
        pltpu.sync_copy(zeros_hbm, acc.at[pl.ds(row0, ROW_SPAN)])
        plsc.subcore_barrier()

        nch = jnp.where(s < (NCHUNK % NS), NCHUNK // NS + 1, NCHUNK // NS)

        def chunk_body(k, _):
            base = half_base + (s + NS * k) * C
            pltpu.sync_copy(src_hbm.at[pl.ds(base, C)], src_v)
            pltpu.sync_copy(dst_hbm.at[pl.ds(base, C)], dst_v)
            pltpu.sync_copy(et_hbm.at[pl.ds(base, C)], et_v)

            @pl.when(c == 0)
            def _():
                pltpu.sync_copy(x0.at[src_v], xb)
                pltpu.sync_copy(r0t.at[et_v], rb)

            @pl.when(c == 1)
            def _():
                pltpu.sync_copy(x1.at[src_v], xb)
                pltpu.sync_copy(r1t.at[et_v], rb)

            def e_body(e, _):
                for j in range(DH // LANES):
                    slr = pl.ds(j * LANES, LANES)
                    sli = pl.ds(DH + j * LANES, LANES)
                    a = xb[e, slr]
                    bb = xb[e, sli]
                    rr = rb[e, slr]
                    ri = rb[e, sli]
                    xb[e, slr] = a * rr - bb * ri
                    xb[e, sli] = a * ri + bb * rr
                return 0
            lax.fori_loop(0, C, e_body, 0)

            pltpu.sync_copy(xb, acc.at[dst_v], add=True)
            return 0

        lax.fori_loop(0, nch, chunk_body, 0)
        plsc.subcore_barrier()
        pltpu.sync_copy(acc.at[pl.ds(row0, ROW_SPAN)],
                        o_ref.at[c, pl.ds(row0, ROW_SPAN)])
        plsc.subcore_barrier()

    run_half(0, out_in)
    run_half(HALF, out_out)


@functools.lru_cache(maxsize=None)
def _sc_agg_kernel():
    return pl.kernel(
        _sc_agg_body,
        out_type=(
            jax.ShapeDtypeStruct((NC, NE, D), jnp.float32),
            jax.ShapeDtypeStruct((NC, NE, D), jnp.float32),
        ),
        mesh=_mesh(),
        scratch_types=[
            pltpu.VMEM_SHARED((NE, D), jnp.float32),
            pltpu.VMEM((C,), jnp.int32),
            pltpu.VMEM((C,), jnp.int32),
            pltpu.VMEM((C,), jnp.int32),
            pltpu.VMEM((C, D), jnp.float32),
            pltpu.VMEM((C, D), jnp.float32),
        ],
    )


NEP = 10240
DRED = NEP // NS


def _sc_deg_body(dst_hbm, out_in, out_out, deg_p, dst_v, stage, rbuf, rsum):
    c = lax.axis_index("c")
    s = lax.axis_index("s")
    base = (c * NS + s) * EPT

    one0 = jnp.where(lax.iota(jnp.int32, LANES) == 0,
                     jnp.float32(1.0), jnp.float32(0.0))

    def _zero(i, _):
        deg_p[pl.ds(i * LANES, LANES)] = jnp.zeros((LANES,), jnp.float32)
        return 0
    lax.fori_loop(0, NEP // LANES, _zero, 0)

    def chunk(k, _):
        pltpu.sync_copy(dst_hbm.at[pl.ds(base + k * DEG_C, DEG_C)], dst_v)

        def grp(g, _):
            dvec = dst_v[pl.ds(g * LANES, LANES)]
            for l in range(LANES):
                d = dvec[l]
                v = deg_p[pl.ds(d, LANES)]
                deg_p[pl.ds(d, LANES)] = v + one0
            return 0
        lax.fori_loop(0, DEG_C // LANES, grp, 0)
        return 0
    lax.fori_loop(0, EPT // DEG_C, chunk, 0)

    pltpu.sync_copy(deg_p, stage.at[s])
    plsc.subcore_barrier()

    pltpu.sync_copy(stage.at[:, pl.ds(s * DRED, DRED)], rbuf)

    def red(g, _):
        sl = pl.ds(g * LANES, LANES)
        acc = rbuf[0, sl]
        for r in range(1, NS):
            acc = acc + rbuf[r, sl]
        rsum[sl] = acc
        return 0
    lax.fori_loop(0, DRED // LANES, red, 0)

    def write(o_ref):
        @pl.when(s < NS - 1)
        def _():
            pltpu.sync_copy(rsum, o_ref.at[pl.ds(s * DRED, DRED)])

        @pl.when(s == NS - 1)
        def _():
            pltpu.sync_copy(rsum.at[pl.ds(0, NE - (NS - 1) * DRED)],
                            o_ref.at[pl.ds((NS - 1) * DRED,
                                           NE - (NS - 1) * DRED)])

    @pl.when(c == 0)
    def _():
        write(out_in)

    @pl.when(c == 1)
    def _():
        write(out_out)


@functools.lru_cache(maxsize=None)
def _sc_deg_kernel():
    return pl.kernel(
        _sc_deg_body,
        out_type=(
            jax.ShapeDtypeStruct((NE,), jnp.float32),
            jax.ShapeDtypeStruct((NE,), jnp.float32),
        ),
        mesh=_mesh(),
        scratch_types=[
            pltpu.VMEM((NEP,), jnp.float32),
            pltpu.VMEM((DEG_C,), jnp.int32),
            pltpu.VMEM_SHARED((NS, NEP), jnp.float32),
            pltpu.VMEM((NS, DRED), jnp.float32),
            pltpu.VMEM((DRED,), jnp.float32),
        ],
    )


def _tc_layer_body(aggi, aggo, degi, dego, xr, xi,
                   win, wout, wloop, lr, li, bb, relr, reli, wrel,
                   yr_o, yi_o, rro_o, rio_o):
    dot = functools.partial(jnp.dot, preferred_element_type=jnp.float32,
                            precision=lax.Precision.HIGHEST)
    ni = 1.0 / jnp.maximum(degi[...], 1.0)
    no = 1.0 / jnp.maximum(dego[...], 1.0)
    air = jnp.concatenate([aggi[0, :, :DH], aggi[1, :, :DH]], axis=-1) * ni
    aii = jnp.concatenate([aggi[0, :, DH:], aggi[1, :, DH:]], axis=-1) * ni
    aor = jnp.concatenate([aggo[0, :, :DH], aggo[1, :, :DH]], axis=-1) * no
    aoi = jnp.concatenate([aggo[0, :, DH:], aggo[1, :, DH:]], axis=-1) * no
    xr_ = xr[...]
    xi_ = xi[...]
    lpr = xr_ * lr[0:1, :] - xi_ * li[0:1, :]
    lpi = xr_ * li[0:1, :] + xi_ * lr[0:1, :]
    yr_o[...] = jnp.tanh((dot(air, win[...]) + dot(aor, wout[...])
                          + dot(lpr, wloop[...])) / 3.0 + bb[0:1, :])
    yi_o[...] = jnp.tanh((dot(aii, win[...]) + dot(aoi, wout[...])
                          + dot(lpi, wloop[...])) / 3.0 + bb[0:1, :])

    @pl.when(pl.program_id(0) == 0)
    def _():
        rro_o[...] = dot(relr[...], wrel[...])
        rio_o[...] = dot(reli[...], wrel[...])


_TC_R = 1000


def _tc_layer(aggi, aggo, degi, dego, xr, xi,
              win, wout, wloop, loop_r, loop_i, b, relr, reli, wrel):
    g = NE // _TC_R
    a_spec = pl.BlockSpec((NC, _TC_R, D), lambda i: (0, i, 0))
    d_spec = pl.BlockSpec((_TC_R, 1), lambda i: (i, 0))
    x_spec = pl.BlockSpec((_TC_R, D), lambda i: (i, 0))
    w_spec = pl.BlockSpec((D, D), lambda i: (0, 0))
    v_spec = pl.BlockSpec((1, D), lambda i: (0, 0))
    r_spec = pl.BlockSpec((NR2, D), lambda i: (0, 0))
    return pl.pallas_call(
        _tc_layer_body,
        grid=(g,),
        in_specs=[a_spec, a_spec, d_spec, d_spec, x_spec, x_spec,
                  w_spec, w_spec, w_spec, v_spec, v_spec, v_spec,
                  r_spec, r_spec, w_spec],
        out_specs=[x_spec, x_spec, r_spec, r_spec],
        out_shape=[
            jax.ShapeDtypeStruct((NE, D), jnp.float32),
            jax.ShapeDtypeStruct((NE, D), jnp.float32),
            jax.ShapeDtypeStruct((NR2, D), jnp.float32),
            jax.ShapeDtypeStruct((NR2, D), jnp.float32),
        ],
    )(aggi, aggo, degi.reshape(NE, 1), dego.reshape(NE, 1),
      xr, xi, win, wout, wloop,
      loop_r.reshape(1, D), loop_i.reshape(1, D), b.reshape(1, D),
      relr, reli, wrel)


BW = B // NW


def _sc_gather_body(heads, tails, rels, rels_t, xr, xi, rr, ri, tt,
                    hxr, hxi, txr, txi, vrr, vri, ht, et2, rt,
                    idx_v, buf_x, buf_t):
    c = lax.axis_index("c")
    s = lax.axis_index("s")
    base = (s * NC + c) * BW

    def gather(idx_hbm, table, buf, out):
        pltpu.sync_copy(idx_hbm.at[pl.ds(base, BW)], idx_v)
        pltpu.sync_copy(table.at[idx_v], buf)
        pltpu.sync_copy(buf, out.at[pl.ds(base, BW)])

    gather(heads, xr, buf_x, hxr)
    gather(heads, xi, buf_x, hxi)
    gather(tails, xr, buf_x, txr)
    gather(tails, xi, buf_x, txi)
    gather(rels, rr, buf_x, vrr)
    gather(rels, ri, buf_x, vri)
    gather(heads, tt, buf_t, ht)
    gather(tails, tt, buf_t, et2)
    gather(rels_t, tt, buf_t, rt)


@functools.lru_cache(maxsize=None)
def _sc_gather_kernel():
    return pl.kernel(
        _sc_gather_body,
        out_type=(
            [jax.ShapeDtypeStruct((B, D), jnp.float32)] * 6
            + [jax.ShapeDtypeStruct((B, TW), jnp.float32)] * 3
        ),
        mesh=_mesh(),
        scratch_types=[
            pltpu.VMEM((BW,), jnp.int32),
            pltpu.VMEM((BW, D), jnp.float32),
            pltpu.VMEM((BW, TW), jnp.float32),
        ],
    )


def _tc_score_body(hxr, hxi, txr, txi, vrr, vri, ht, tt2, rt, y, m, d, out):
    def temb(tref, f):
        t = tref[...]
        parts = [t[:, j * DT:(j + 1) * DT] for j in range(9)]
        ya, ma, da, yf, mf, df, yp, mp, dp = parts
        return (ya * f(yf * y[...] + yp) + ma * f(mf * m[...] + mp)
                + da * f(df * d[...] + dp))

    thr = temb(ht, jnp.cos)
    thi = temb(ht, jnp.sin)
    ttr = temb(tt2, jnp.cos)
    tti = temb(tt2, jnp.sin)
    trr = temb(rt, jnp.cos)
    tri = temb(rt, jnp.sin)

    sx = jnp.sum(vrr[...] * (hxr[...] * txr[...] + hxi[...] * txi[...])
                 + vri[...] * (hxr[...] * txi[...] - hxi[...] * txr[...]),
                 axis=1, keepdims=True)
    st = jnp.sum(trr * (thr * ttr + thi * tti) + tri * (thr * tti - thi * ttr),
                 axis=1, keepdims=True)
    out[...] = sx + st


_SC_R = 256


def _tc_score(gathered, y, m, d):
    g = B // _SC_R
    x_spec = pl.BlockSpec((_SC_R, D), lambda i: (i, 0))
    t_spec = pl.BlockSpec((_SC_R, TW), lambda i: (i, 0))
    s_spec = pl.BlockSpec((_SC_R, 1), lambda i: (i, 0))
    return pl.pallas_call(
        _tc_score_body,
        grid=(g,),
        in_specs=[x_spec] * 6 + [t_spec] * 3 + [s_spec] * 3,
        out_specs=s_spec,
        out_shape=jax.ShapeDtypeStruct((B, 1), jnp.float32),
    )(*gathered, y.reshape(B, 1), m.reshape(B, 1), d.reshape(B, 1))


def _combine(re, im):
    return (jnp.concatenate([re[:, :DH], im[:, :DH]], axis=1),
            jnp.concatenate([re[:, DH:], im[:, DH:]], axis=1))


def kernel(heads, relations, tails, years, months, days, intervals,
           edge_index, edge_type, ent_r, ent_i, rel_r, rel_i,
           y_amp, m_amp, d_amp, y_freq, m_freq, d_freq, y_phi, m_phi, d_phi,
           W_in1, W_out1, W_loop1, w_rel1, b1, loop_r1, loop_i1,
           W_in2, W_out2, W_loop2, w_rel2, b2, loop_r2, loop_i2):
    src = edge_index[0]
    dst = edge_index[1]
    et = edge_type

    dgi, dgo = _sc_deg_kernel()(dst)

    zrows = jnp.zeros((ROW_SPAN, D), jnp.float32)
    x0, x1 = _combine(ent_r, ent_i)
    r0t, r1t = _combine(rel_r, rel_i)
    aggi, aggo = _sc_agg_kernel()(src, dst, et, x0, x1, r0t, r1t, zrows)
    yr1, yi1, rrn1, rin1 = _tc_layer(
        aggi, aggo, dgi, dgo, ent_r, ent_i,
        W_in1, W_out1, W_loop1, loop_r1, loop_i1, b1, rel_r, rel_i, w_rel1)

    x0, x1 = _combine(yr1, yi1)
    r0t, r1t = _combine(rrn1, rin1)
    aggi, aggo = _sc_agg_kernel()(src, dst, et, x0, x1, r0t, r1t, zrows)
    yr2, yi2, rrn2, rin2 = _tc_layer(
        aggi, aggo, dgi, dgo, yr1, yi1,
        W_in2, W_out2, W_loop2, loop_r2, loop_i2, b2, rrn1, rin1, w_rel2)

    tt = jnp.concatenate(
        [y_amp, m_amp, d_amp, y_freq, m_freq, d_freq, y_phi, m_phi, d_phi,
         jnp.zeros((NT, TW - 9 * DT), jnp.float32)], axis=1)
    gathered = _sc_gather_kernel()(
        heads, tails, relations, relations + NE, yr2, yi2, rrn2, rin2, tt)
    scores = _tc_score(gathered, years, months, days)
    return scores[:, 0]


# speedup vs baseline: 7.3933x; 1.7310x over previous
import functools

import jax
import jax.numpy as jnp
from jax import lax
from jax.experimental import pallas as pl
from jax.experimental.pallas import tpu as pltpu
from jax.experimental.pallas import tpu_sc as plsc

NE = 10000
NR2 = 400
D = 128
DH = 64
DT = 64
NT = NE + NR2
TW = 9 * DT + DT
E = 320000
HALF = E // 2
B = 1024

NC = 2
NS = 16
NW = NC * NS
LANES = 16

C = 80
NCHUNK = HALF // C
TNCH = NCHUNK // NS
PAIRS = (TNCH - 1) // 2
ROW_STRIDE = 624
ROW_SPAN = 640
ZROWS = 160

EPT = E // NW
DEG_C = 2000


@functools.lru_cache(maxsize=None)
def _mesh():
    return plsc.VectorSubcoreMesh(core_axis_name="c", subcore_axis_name="s",
                                  num_cores=NC, num_subcores=NS)


def _sc_agg_body(src_hbm, dst_hbm, et_hbm, xcat, rcat, zeros_hbm,
                 out_in, out_out,
                 acc, sv0, sv1, dv0, dv1, ev0, ev1, dvs0, dvs1,
                 xb0, xb1, rb0, rb1,
                 i0, i1, g0, g1, ss0, ss1):
    c = lax.axis_index("c")
    s = lax.axis_index("s")
    row0 = s * ROW_STRIDE
    sv = (sv0, sv1)
    dv = (dv0, dv1)
    ev = (ev0, ev1)
    dvs = (dvs0, dvs1)
    xb = (xb0, xb1)
    rb = (rb0, rb1)
    isem = (i0, i1)
    gsem = (g0, g1)
    ssem = (ss0, ss1)
    xoff = c * NE
    roff = c * NR2

    def fire_idx(b, base):
        pltpu.async_copy(src_hbm.at[pl.ds(base, C)], sv[b], isem[b])
        pltpu.async_copy(dst_hbm.at[pl.ds(base, C)], dv[b], isem[b])
        pltpu.async_copy(et_hbm.at[pl.ds(base, C)], ev[b], isem[b])

    def wait_idx(b):
        for ref, hbm in ((sv[b], src_hbm), (dv[b], dst_hbm), (ev[b], et_hbm)):
            pltpu.make_async_copy(hbm.at[pl.ds(0, C)], ref, isem[b]).wait()
        for g in range(C // LANES):
            sl = pl.ds(g * LANES, LANES)
            sv[b][sl] = sv[b][sl] + xoff
            ev[b][sl] = ev[b][sl] + roff
            dvs[b][sl] = dv[b][sl]

    def fire_g(b):
        pltpu.async_copy(xcat.at[sv[b]], xb[b], gsem[b])
        pltpu.async_copy(rcat.at[ev[b]], rb[b], gsem[b])

    def wait_g(b):
        pltpu.make_async_copy(xcat.at[sv[b]], xb[b], gsem[b]).wait()
        pltpu.make_async_copy(rcat.at[ev[b]], rb[b], gsem[b]).wait()

    def compute(b):
        xbb, rbb = xb[b], rb[b]

        def e_body(e, _):
            for j in range(DH // LANES):
                slr = pl.ds(j * LANES, LANES)
                sli = pl.ds(DH + j * LANES, LANES)
                a = xbb[e, slr]
                bb = xbb[e, sli]
                rr = rbb[e, slr]
                ri = rbb[e, sli]
                xbb[e, slr] = a * rr - bb * ri
                xbb[e, sli] = a * ri + bb * rr
            return 0
        lax.fori_loop(0, C, e_body, 0)

    def fire_s(b):
        pltpu.async_copy(xb[b], acc.at[dvs[b]], ssem[b], add=True)

    def wait_s(b):
        pltpu.make_async_copy(xb[b], acc.at[dvs[b]], ssem[b]).wait()

    def run_half(half_base, o_ref):
        pltpu.sync_copy(zeros_hbm, acc.at[pl.ds(row0, ROW_SPAN)])
        plsc.subcore_barrier()

        def chbase(k):
            return half_base + (s + NS * k) * C

        fire_idx(0, chbase(0))
        wait_idx(0)
        fire_g(0)
        fire_idx(1, chbase(1))

        def pair(p, _):
            @pl.when(p > 0)
            def _():
                wait_s(1)
            wait_idx(1)
            fire_g(1)
            wait_g(0)
            fire_idx(0, chbase(2 * p + 2))
            compute(0)
            fire_s(0)
            wait_g(1)

            @pl.when(p < PAIRS - 1)
            def _():
                fire_idx(1, chbase(2 * p + 3))
            compute(1)
            fire_s(1)
            wait_s(0)
            wait_idx(0)
            fire_g(0)
            return 0

        lax.fori_loop(0, PAIRS, pair, 0)
        wait_g(0)
        compute(0)
        fire_s(0)
        wait_s(0)
        wait_s(1)

        plsc.subcore_barrier()
        pltpu.sync_copy(acc.at[pl.ds(row0, ROW_SPAN)],
                        o_ref.at[c, pl.ds(row0, ROW_SPAN)])
        plsc.subcore_barrier()

    run_half(0, out_in)
    run_half(HALF, out_out)


@functools.lru_cache(maxsize=None)
def _sc_agg_kernel():
    return pl.kernel(
        _sc_agg_body,
        out_type=(
            jax.ShapeDtypeStruct((NC, NE, D), jnp.float32),
            jax.ShapeDtypeStruct((NC, NE, D), jnp.float32),
        ),
        mesh=_mesh(),
        scratch_types=[
            pltpu.VMEM_SHARED((NE, D), jnp.float32),
            pltpu.VMEM((C,), jnp.int32),
            pltpu.VMEM((C,), jnp.int32),
            pltpu.VMEM((C,), jnp.int32),
            pltpu.VMEM((C,), jnp.int32),
            pltpu.VMEM((C,), jnp.int32),
            pltpu.VMEM((C,), jnp.int32),
            pltpu.VMEM((C,), jnp.int32),
            pltpu.VMEM((C,), jnp.int32),
            pltpu.VMEM((C, D), jnp.float32),
            pltpu.VMEM((C, D), jnp.float32),
            pltpu.VMEM((C, D), jnp.float32),
            pltpu.VMEM((C, D), jnp.float32),
            pltpu.SemaphoreType.DMA,
            pltpu.SemaphoreType.DMA,
            pltpu.SemaphoreType.DMA,
            pltpu.SemaphoreType.DMA,
            pltpu.SemaphoreType.DMA,
            pltpu.SemaphoreType.DMA,
        ],
    )


NEP = 10240
DRED = NEP // NS


def _sc_deg_body(dst_hbm, out_in, out_out, deg_p, dst_v, stage, rbuf, rsum):
    c = lax.axis_index("c")
    s = lax.axis_index("s")
    base = (c * NS + s) * EPT

    one0 = jnp.where(lax.iota(jnp.int32, LANES) == 0,
                     jnp.float32(1.0), jnp.float32(0.0))

    def _zero(i, _):
        deg_p[pl.ds(i * LANES, LANES)] = jnp.zeros((LANES,), jnp.float32)
        return 0
    lax.fori_loop(0, NEP // LANES, _zero, 0)

    def chunk(k, _):
        pltpu.sync_copy(dst_hbm.at[pl.ds(base + k * DEG_C, DEG_C)], dst_v)

        def grp(g, _):
            dvec = dst_v[pl.ds(g * LANES, LANES)]
            for l in range(LANES):
                d = dvec[l]
                v = deg_p[pl.ds(d, LANES)]
                deg_p[pl.ds(d, LANES)] = v + one0
            return 0
        lax.fori_loop(0, DEG_C // LANES, grp, 0)
        return 0
    lax.fori_loop(0, EPT // DEG_C, chunk, 0)

    pltpu.sync_copy(deg_p, stage.at[s])
    plsc.subcore_barrier()

    pltpu.sync_copy(stage.at[:, pl.ds(s * DRED, DRED)], rbuf)

    def red(g, _):
        sl = pl.ds(g * LANES, LANES)
        acc = rbuf[0, sl]
        for r in range(1, NS):
            acc = acc + rbuf[r, sl]
        rsum[sl] = acc
        return 0
    lax.fori_loop(0, DRED // LANES, red, 0)

    def write(o_ref):
        @pl.when(s < NS - 1)
        def _():
            pltpu.sync_copy(rsum, o_ref.at[pl.ds(s * DRED, DRED)])

        @pl.when(s == NS - 1)
        def _():
            pltpu.sync_copy(rsum.at[pl.ds(0, NE - (NS - 1) * DRED)],
                            o_ref.at[pl.ds((NS - 1) * DRED,
                                           NE - (NS - 1) * DRED)])

    @pl.when(c == 0)
    def _():
        write(out_in)

    @pl.when(c == 1)
    def _():
        write(out_out)


@functools.lru_cache(maxsize=None)
def _sc_deg_kernel():
    return pl.kernel(
        _sc_deg_body,
        out_type=(
            jax.ShapeDtypeStruct((NE,), jnp.float32),
            jax.ShapeDtypeStruct((NE,), jnp.float32),
        ),
        mesh=_mesh(),
        scratch_types=[
            pltpu.VMEM((NEP,), jnp.float32),
            pltpu.VMEM((DEG_C,), jnp.int32),
            pltpu.VMEM_SHARED((NS, NEP), jnp.float32),
            pltpu.VMEM((NS, DRED), jnp.float32),
            pltpu.VMEM((DRED,), jnp.float32),
        ],
    )


def _tc_layer_body(aggi, aggo, degi, dego, xr, xi,
                   win, wout, wloop, lr, li, bb, relr, reli, wrel,
                   yr_o, yi_o, rro_o, rio_o):
    dot = functools.partial(jnp.dot, preferred_element_type=jnp.float32,
                            precision=lax.Precision.HIGHEST)
    ni = 1.0 / jnp.maximum(degi[...], 1.0)
    no = 1.0 / jnp.maximum(dego[...], 1.0)
    air = jnp.concatenate([aggi[0, :, :DH], aggi[1, :, :DH]], axis=-1) * ni
    aii = jnp.concatenate([aggi[0, :, DH:], aggi[1, :, DH:]], axis=-1) * ni
    aor = jnp.concatenate([aggo[0, :, :DH], aggo[1, :, :DH]], axis=-1) * no
    aoi = jnp.concatenate([aggo[0, :, DH:], aggo[1, :, DH:]], axis=-1) * no
    xr_ = xr[...]
    xi_ = xi[...]
    lpr = xr_ * lr[0:1, :] - xi_ * li[0:1, :]
    lpi = xr_ * li[0:1, :] + xi_ * lr[0:1, :]
    yr_o[...] = jnp.tanh((dot(air, win[...]) + dot(aor, wout[...])
                          + dot(lpr, wloop[...])) / 3.0 + bb[0:1, :])
    yi_o[...] = jnp.tanh((dot(aii, win[...]) + dot(aoi, wout[...])
                          + dot(lpi, wloop[...])) / 3.0 + bb[0:1, :])

    @pl.when(pl.program_id(0) == 0)
    def _():
        rro_o[...] = dot(relr[...], wrel[...])
        rio_o[...] = dot(reli[...], wrel[...])


_TC_R = 1000


def _tc_layer(aggi, aggo, degi, dego, xr, xi,
              win, wout, wloop, loop_r, loop_i, b, relr, reli, wrel):
    g = NE // _TC_R
    a_spec = pl.BlockSpec((NC, _TC_R, D), lambda i: (0, i, 0))
    d_spec = pl.BlockSpec((_TC_R, 1), lambda i: (i, 0))
    x_spec = pl.BlockSpec((_TC_R, D), lambda i: (i, 0))
    w_spec = pl.BlockSpec((D, D), lambda i: (0, 0))
    v_spec = pl.BlockSpec((1, D), lambda i: (0, 0))
    r_spec = pl.BlockSpec((NR2, D), lambda i: (0, 0))
    return pl.pallas_call(
        _tc_layer_body,
        grid=(g,),
        in_specs=[a_spec, a_spec, d_spec, d_spec, x_spec, x_spec,
                  w_spec, w_spec, w_spec, v_spec, v_spec, v_spec,
                  r_spec, r_spec, w_spec],
        out_specs=[x_spec, x_spec, r_spec, r_spec],
        out_shape=[
            jax.ShapeDtypeStruct((NE, D), jnp.float32),
            jax.ShapeDtypeStruct((NE, D), jnp.float32),
            jax.ShapeDtypeStruct((NR2, D), jnp.float32),
            jax.ShapeDtypeStruct((NR2, D), jnp.float32),
        ],
    )(aggi, aggo, degi.reshape(NE, 1), dego.reshape(NE, 1),
      xr, xi, win, wout, wloop,
      loop_r.reshape(1, D), loop_i.reshape(1, D), b.reshape(1, D),
      relr, reli, wrel)


BW = B // NW


def _sc_gather_body(heads, tails, rels, rels_t, xr, xi, rr, ri, tt,
                    hxr, hxi, txr, txi, vrr, vri, ht, et2, rt,
                    idx_v, buf_x, buf_t):
    c = lax.axis_index("c")
    s = lax.axis_index("s")
    base = (s * NC + c) * BW

    def gather(idx_hbm, table, buf, out):
        pltpu.sync_copy(idx_hbm.at[pl.ds(base, BW)], idx_v)
        pltpu.sync_copy(table.at[idx_v], buf)
        pltpu.sync_copy(buf, out.at[pl.ds(base, BW)])

    gather(heads, xr, buf_x, hxr)
    gather(heads, xi, buf_x, hxi)
    gather(tails, xr, buf_x, txr)
    gather(tails, xi, buf_x, txi)
    gather(rels, rr, buf_x, vrr)
    gather(rels, ri, buf_x, vri)
    gather(heads, tt, buf_t, ht)
    gather(tails, tt, buf_t, et2)
    gather(rels_t, tt, buf_t, rt)


@functools.lru_cache(maxsize=None)
def _sc_gather_kernel():
    return pl.kernel(
        _sc_gather_body,
        out_type=(
            [jax.ShapeDtypeStruct((B, D), jnp.float32)] * 6
            + [jax.ShapeDtypeStruct((B, TW), jnp.float32)] * 3
        ),
        mesh=_mesh(),
        scratch_types=[
            pltpu.VMEM((BW,), jnp.int32),
            pltpu.VMEM((BW, D), jnp.float32),
            pltpu.VMEM((BW, TW), jnp.float32),
        ],
    )


def _tc_score_body(hxr, hxi, txr, txi, vrr, vri, ht, tt2, rt, y, m, d, out):
    def temb(tref, f):
        t = tref[...]
        parts = [t[:, j * DT:(j + 1) * DT] for j in range(9)]
        ya, ma, da, yf, mf, df, yp, mp, dp = parts
        return (ya * f(yf * y[...] + yp) + ma * f(mf * m[...] + mp)
                + da * f(df * d[...] + dp))

    thr = temb(ht, jnp.cos)
    thi = temb(ht, jnp.sin)
    ttr = temb(tt2, jnp.cos)
    tti = temb(tt2, jnp.sin)
    trr = temb(rt, jnp.cos)
    tri = temb(rt, jnp.sin)

    sx = jnp.sum(vrr[...] * (hxr[...] * txr[...] + hxi[...] * txi[...])
                 + vri[...] * (hxr[...] * txi[...] - hxi[...] * txr[...]),
                 axis=1, keepdims=True)
    st = jnp.sum(trr * (thr * ttr + thi * tti) + tri * (thr * tti - thi * ttr),
                 axis=1, keepdims=True)
    out[...] = sx + st


_SC_R = 256


def _tc_score(gathered, y, m, d):
    g = B // _SC_R
    x_spec = pl.BlockSpec((_SC_R, D), lambda i: (i, 0))
    t_spec = pl.BlockSpec((_SC_R, TW), lambda i: (i, 0))
    s_spec = pl.BlockSpec((_SC_R, 1), lambda i: (i, 0))
    return pl.pallas_call(
        _tc_score_body,
        grid=(g,),
        in_specs=[x_spec] * 6 + [t_spec] * 3 + [s_spec] * 3,
        out_specs=s_spec,
        out_shape=jax.ShapeDtypeStruct((B, 1), jnp.float32),
    )(*gathered, y.reshape(B, 1), m.reshape(B, 1), d.reshape(B, 1))


def _combine(re, im):
    return jnp.concatenate(
        [jnp.concatenate([re[:, :DH], im[:, :DH]], axis=1),
         jnp.concatenate([re[:, DH:], im[:, DH:]], axis=1)], axis=0)


def kernel(heads, relations, tails, years, months, days, intervals,
           edge_index, edge_type, ent_r, ent_i, rel_r, rel_i,
           y_amp, m_amp, d_amp, y_freq, m_freq, d_freq, y_phi, m_phi, d_phi,
           W_in1, W_out1, W_loop1, w_rel1, b1, loop_r1, loop_i1,
           W_in2, W_out2, W_loop2, w_rel2, b2, loop_r2, loop_i2):
    src = edge_index[0]
    dst = edge_index[1]
    et = edge_type

    dgi, dgo = _sc_deg_kernel()(dst)

    zrows = jnp.zeros((ROW_SPAN, D), jnp.float32)
    xcat = _combine(ent_r, ent_i)
    rcat = _combine(rel_r, rel_i)
    aggi, aggo = _sc_agg_kernel()(src, dst, et, xcat, rcat, zrows)
    yr1, yi1, rrn1, rin1 = _tc_layer(
        aggi, aggo, dgi, dgo, ent_r, ent_i,
        W_in1, W_out1, W_loop1, loop_r1, loop_i1, b1, rel_r, rel_i, w_rel1)

    xcat = _combine(yr1, yi1)
    rcat = _combine(rrn1, rin1)
    aggi, aggo = _sc_agg_kernel()(src, dst, et, xcat, rcat, zrows)
    yr2, yi2, rrn2, rin2 = _tc_layer(
        aggi, aggo, dgi, dgo, yr1, yi1,
        W_in2, W_out2, W_loop2, loop_r2, loop_i2, b2, rrn1, rin1, w_rel2)

    tt = jnp.concatenate(
        [y_amp, m_amp, d_amp, y_freq, m_freq, d_freq, y_phi, m_phi, d_phi,
         jnp.zeros((NT, TW - 9 * DT), jnp.float32)], axis=1)
    gathered = _sc_gather_kernel()(
        heads, tails, relations, relations + NE, yr2, yi2, rrn2, rin2, tt)
    scores = _tc_score(gathered, years, months, days)
    return scores[:, 0]


# speedup vs baseline: 7.8137x; 1.0569x over previous
import functools

import jax
import jax.numpy as jnp
from jax import lax
from jax.experimental import pallas as pl
from jax.experimental.pallas import tpu as pltpu
from jax.experimental.pallas import tpu_sc as plsc

NE = 10000
NR2 = 400
D = 128
DH = 64
DT = 64
NT = NE + NR2
TW = 9 * DT + DT
E = 320000
HALF = E // 2
B = 1024

NC = 2
NS = 16
NW = NC * NS
LANES = 16

C = 80
NCHUNK = HALF // C
TNCH = NCHUNK // NS
PAIRS = (TNCH - 1) // 2
ROW_STRIDE = 624
ROW_SPAN = 640
ZROWS = 160

EPT = E // NW
DEG_C = 2000


@functools.lru_cache(maxsize=None)
def _mesh():
    return plsc.VectorSubcoreMesh(core_axis_name="c", subcore_axis_name="s",
                                  num_cores=NC, num_subcores=NS)


def _sc_agg_body(src_hbm, dst_hbm, et_hbm, xcat, rcat, zeros_hbm,
                 out_in, out_out,
                 acc, sv0, sv1, dv0, dv1, ev0, ev1, dvs0, dvs1,
                 xb0, xb1, rb0, rb1,
                 i0, i1, g0, g1, ss0, ss1):
    c = lax.axis_index("c")
    s = lax.axis_index("s")
    row0 = s * ROW_STRIDE
    sv = (sv0, sv1)
    dv = (dv0, dv1)
    ev = (ev0, ev1)
    dvs = (dvs0, dvs1)
    xb = (xb0, xb1)
    rb = (rb0, rb1)
    isem = (i0, i1)
    gsem = (g0, g1)
    ssem = (ss0, ss1)
    xoff = c * NE
    roff = c * NR2

    def fire_idx(b, base):
        pltpu.async_copy(src_hbm.at[pl.ds(base, C)], sv[b], isem[b])
        pltpu.async_copy(dst_hbm.at[pl.ds(base, C)], dv[b], isem[b])
        pltpu.async_copy(et_hbm.at[pl.ds(base, C)], ev[b], isem[b])

    def wait_idx(b):
        for ref, hbm in ((sv[b], src_hbm), (dv[b], dst_hbm), (ev[b], et_hbm)):
            pltpu.make_async_copy(hbm.at[pl.ds(0, C)], ref, isem[b]).wait()
        for g in range(C // LANES):
            sl = pl.ds(g * LANES, LANES)
            sv[b][sl] = sv[b][sl] + xoff
            ev[b][sl] = ev[b][sl] + roff
            dvs[b][sl] = dv[b][sl]

    def fire_g(b):
        pltpu.async_copy(xcat.at[sv[b]], xb[b], gsem[b])
        pltpu.async_copy(rcat.at[ev[b]], rb[b], gsem[b])

    def wait_g(b):
        pltpu.make_async_copy(xcat.at[sv[b]], xb[b], gsem[b]).wait()
        pltpu.make_async_copy(rcat.at[ev[b]], rb[b], gsem[b]).wait()

    def compute(b):
        xbb, rbb = xb[b], rb[b]

        def e_body(e, _):
            for j in range(DH // LANES):
                slr = pl.ds(j * LANES, LANES)
                sli = pl.ds(DH + j * LANES, LANES)
                a = xbb[e, slr]
                bb = xbb[e, sli]
                rr = rbb[e, slr]
                ri = rbb[e, sli]
                xbb[e, slr] = a * rr - bb * ri
                xbb[e, sli] = a * ri + bb * rr
            return 0
        lax.fori_loop(0, C, e_body, 0)

    def fire_s(b):
        pltpu.async_copy(xb[b], acc.at[dvs[b]], ssem[b], add=True)

    def wait_s(b):
        pltpu.make_async_copy(xb[b], acc.at[dvs[b]], ssem[b]).wait()

    def run_half(half_base, o_ref):
        pltpu.sync_copy(zeros_hbm, acc.at[pl.ds(row0, ROW_SPAN)])
        plsc.subcore_barrier()

        def chbase(k):
            return half_base + (s + NS * k) * C

        fire_idx(0, chbase(0))
        wait_idx(0)
        fire_g(0)
        fire_idx(1, chbase(1))

        def pair(p, _):
            @pl.when(p > 0)
            def _():
                wait_s(1)
            wait_idx(1)
            fire_g(1)
            wait_g(0)
            fire_idx(0, chbase(2 * p + 2))
            compute(0)
            fire_s(0)
            wait_g(1)

            @pl.when(p < PAIRS - 1)
            def _():
                fire_idx(1, chbase(2 * p + 3))
            compute(1)
            fire_s(1)
            wait_s(0)
            wait_idx(0)
            fire_g(0)
            return 0

        lax.fori_loop(0, PAIRS, pair, 0)
        wait_g(0)
        compute(0)
        fire_s(0)
        wait_s(0)
        wait_s(1)

        plsc.subcore_barrier()
        pltpu.sync_copy(acc.at[pl.ds(row0, ROW_SPAN)],
                        o_ref.at[c, pl.ds(row0, ROW_SPAN)])
        plsc.subcore_barrier()

    run_half(0, out_in)
    run_half(HALF, out_out)


@functools.lru_cache(maxsize=None)
def _sc_agg_kernel():
    return pl.kernel(
        _sc_agg_body,
        out_type=(
            jax.ShapeDtypeStruct((NC, NE, D), jnp.float32),
            jax.ShapeDtypeStruct((NC, NE, D), jnp.float32),
        ),
        mesh=_mesh(),
        scratch_types=[
            pltpu.VMEM_SHARED((NE, D), jnp.float32),
            pltpu.VMEM((C,), jnp.int32),
            pltpu.VMEM((C,), jnp.int32),
            pltpu.VMEM((C,), jnp.int32),
            pltpu.VMEM((C,), jnp.int32),
            pltpu.VMEM((C,), jnp.int32),
            pltpu.VMEM((C,), jnp.int32),
            pltpu.VMEM((C,), jnp.int32),
            pltpu.VMEM((C,), jnp.int32),
            pltpu.VMEM((C, D), jnp.float32),
            pltpu.VMEM((C, D), jnp.float32),
            pltpu.VMEM((C, D), jnp.float32),
            pltpu.VMEM((C, D), jnp.float32),
            pltpu.SemaphoreType.DMA,
            pltpu.SemaphoreType.DMA,
            pltpu.SemaphoreType.DMA,
            pltpu.SemaphoreType.DMA,
            pltpu.SemaphoreType.DMA,
            pltpu.SemaphoreType.DMA,
        ],
    )


NEP = 10240
DRED = NEP // NS


def _sc_deg_body(dst_hbm, out_in, out_out, deg_p, dst_v, stage, rbuf, rsum):
    c = lax.axis_index("c")
    s = lax.axis_index("s")
    base = (c * NS + s) * EPT

    one0 = jnp.where(lax.iota(jnp.int32, LANES) == 0,
                     jnp.float32(1.0), jnp.float32(0.0))

    def _zero(i, _):
        deg_p[pl.ds(i * LANES, LANES)] = jnp.zeros((LANES,), jnp.float32)
        return 0
    lax.fori_loop(0, NEP // LANES, _zero, 0)

    def chunk(k, _):
        pltpu.sync_copy(dst_hbm.at[pl.ds(base + k * DEG_C, DEG_C)], dst_v)

        def grp(g, _):
            dvec = dst_v[pl.ds(g * LANES, LANES)]
            for l in range(LANES):
                d = dvec[l]
                v = deg_p[pl.ds(d, LANES)]
                deg_p[pl.ds(d, LANES)] = v + one0
            return 0
        lax.fori_loop(0, DEG_C // LANES, grp, 0)
        return 0
    lax.fori_loop(0, EPT // DEG_C, chunk, 0)

    pltpu.sync_copy(deg_p, stage.at[s])
    plsc.subcore_barrier()

    pltpu.sync_copy(stage.at[:, pl.ds(s * DRED, DRED)], rbuf)

    def red(g, _):
        sl = pl.ds(g * LANES, LANES)
        acc = rbuf[0, sl]
        for r in range(1, NS):
            acc = acc + rbuf[r, sl]
        rsum[sl] = acc
        return 0
    lax.fori_loop(0, DRED // LANES, red, 0)

    def write(o_ref):
        @pl.when(s < NS - 1)
        def _():
            pltpu.sync_copy(rsum, o_ref.at[pl.ds(s * DRED, DRED)])

        @pl.when(s == NS - 1)
        def _():
            pltpu.sync_copy(rsum.at[pl.ds(0, NE - (NS - 1) * DRED)],
                            o_ref.at[pl.ds((NS - 1) * DRED,
                                           NE - (NS - 1) * DRED)])

    @pl.when(c == 0)
    def _():
        write(out_in)

    @pl.when(c == 1)
    def _():
        write(out_out)


@functools.lru_cache(maxsize=None)
def _sc_deg_kernel():
    return pl.kernel(
        _sc_deg_body,
        out_type=(
            jax.ShapeDtypeStruct((NE,), jnp.float32),
            jax.ShapeDtypeStruct((NE,), jnp.float32),
        ),
        mesh=_mesh(),
        scratch_types=[
            pltpu.VMEM((NEP,), jnp.float32),
            pltpu.VMEM((DEG_C,), jnp.int32),
            pltpu.VMEM_SHARED((NS, NEP), jnp.float32),
            pltpu.VMEM((NS, DRED), jnp.float32),
            pltpu.VMEM((DRED,), jnp.float32),
        ],
    )


def _tc_layer_body(aggi, aggo, degi, dego, xr, xi,
                   win, wout, wloop, lr, li, bb, relr, reli, wrel,
                   yr_o, yi_o, rro_o, rio_o):
    dot = functools.partial(jnp.dot, preferred_element_type=jnp.float32)
    ni = 1.0 / jnp.maximum(degi[...], 1.0)
    no = 1.0 / jnp.maximum(dego[...], 1.0)
    air = jnp.concatenate([aggi[0, :, :DH], aggi[1, :, :DH]], axis=-1) * ni
    aii = jnp.concatenate([aggi[0, :, DH:], aggi[1, :, DH:]], axis=-1) * ni
    aor = jnp.concatenate([aggo[0, :, :DH], aggo[1, :, :DH]], axis=-1) * no
    aoi = jnp.concatenate([aggo[0, :, DH:], aggo[1, :, DH:]], axis=-1) * no
    xr_ = xr[...]
    xi_ = xi[...]
    lpr = xr_ * lr[0:1, :] - xi_ * li[0:1, :]
    lpi = xr_ * li[0:1, :] + xi_ * lr[0:1, :]
    yr_o[...] = jnp.tanh((dot(air, win[...]) + dot(aor, wout[...])
                          + dot(lpr, wloop[...])) / 3.0 + bb[0:1, :])
    yi_o[...] = jnp.tanh((dot(aii, win[...]) + dot(aoi, wout[...])
                          + dot(lpi, wloop[...])) / 3.0 + bb[0:1, :])

    @pl.when(pl.program_id(0) == 0)
    def _():
        rro_o[...] = dot(relr[...], wrel[...])
        rio_o[...] = dot(reli[...], wrel[...])


_TC_R = 1000


def _tc_layer(aggi, aggo, degi, dego, xr, xi,
              win, wout, wloop, loop_r, loop_i, b, relr, reli, wrel):
    g = NE // _TC_R
    a_spec = pl.BlockSpec((NC, _TC_R, D), lambda i: (0, i, 0))
    d_spec = pl.BlockSpec((_TC_R, 1), lambda i: (i, 0))
    x_spec = pl.BlockSpec((_TC_R, D), lambda i: (i, 0))
    w_spec = pl.BlockSpec((D, D), lambda i: (0, 0))
    v_spec = pl.BlockSpec((1, D), lambda i: (0, 0))
    r_spec = pl.BlockSpec((NR2, D), lambda i: (0, 0))
    return pl.pallas_call(
        _tc_layer_body,
        grid=(g,),
        in_specs=[a_spec, a_spec, d_spec, d_spec, x_spec, x_spec,
                  w_spec, w_spec, w_spec, v_spec, v_spec, v_spec,
                  r_spec, r_spec, w_spec],
        out_specs=[x_spec, x_spec, r_spec, r_spec],
        out_shape=[
            jax.ShapeDtypeStruct((NE, D), jnp.float32),
            jax.ShapeDtypeStruct((NE, D), jnp.float32),
            jax.ShapeDtypeStruct((NR2, D), jnp.float32),
            jax.ShapeDtypeStruct((NR2, D), jnp.float32),
        ],
    )(aggi, aggo, degi.reshape(NE, 1), dego.reshape(NE, 1),
      xr, xi, win, wout, wloop,
      loop_r.reshape(1, D), loop_i.reshape(1, D), b.reshape(1, D),
      relr, reli, wrel)


BW = B // NW


def _sc_gather_body(heads, tails, rels, rels_t, xr, xi, rr, ri, tt,
                    hxr, hxi, txr, txi, vrr, vri, ht, et2, rt,
                    idx_v, buf_x, buf_t):
    c = lax.axis_index("c")
    s = lax.axis_index("s")
    base = (s * NC + c) * BW

    def gather(idx_hbm, table, buf, out):
        pltpu.sync_copy(idx_hbm.at[pl.ds(base, BW)], idx_v)
        pltpu.sync_copy(table.at[idx_v], buf)
        pltpu.sync_copy(buf, out.at[pl.ds(base, BW)])

    gather(heads, xr, buf_x, hxr)
    gather(heads, xi, buf_x, hxi)
    gather(tails, xr, buf_x, txr)
    gather(tails, xi, buf_x, txi)
    gather(rels, rr, buf_x, vrr)
    gather(rels, ri, buf_x, vri)
    gather(heads, tt, buf_t, ht)
    gather(tails, tt, buf_t, et2)
    gather(rels_t, tt, buf_t, rt)


@functools.lru_cache(maxsize=None)
def _sc_gather_kernel():
    return pl.kernel(
        _sc_gather_body,
        out_type=(
            [jax.ShapeDtypeStruct((B, D), jnp.float32)] * 6
            + [jax.ShapeDtypeStruct((B, TW), jnp.float32)] * 3
        ),
        mesh=_mesh(),
        scratch_types=[
            pltpu.VMEM((BW,), jnp.int32),
            pltpu.VMEM((BW, D), jnp.float32),
            pltpu.VMEM((BW, TW), jnp.float32),
        ],
    )


def _tc_score_body(hxr, hxi, txr, txi, vrr, vri, ht, tt2, rt, y, m, d, out):
    def temb(tref, f):
        t = tref[...]
        parts = [t[:, j * DT:(j + 1) * DT] for j in range(9)]
        ya, ma, da, yf, mf, df, yp, mp, dp = parts
        return (ya * f(yf * y[...] + yp) + ma * f(mf * m[...] + mp)
                + da * f(df * d[...] + dp))

    thr = temb(ht, jnp.cos)
    thi = temb(ht, jnp.sin)
    ttr = temb(tt2, jnp.cos)
    tti = temb(tt2, jnp.sin)
    trr = temb(rt, jnp.cos)
    tri = temb(rt, jnp.sin)

    sx = jnp.sum(vrr[...] * (hxr[...] * txr[...] + hxi[...] * txi[...])
                 + vri[...] * (hxr[...] * txi[...] - hxi[...] * txr[...]),
                 axis=1, keepdims=True)
    st = jnp.sum(trr * (thr * ttr + thi * tti) + tri * (thr * tti - thi * ttr),
                 axis=1, keepdims=True)
    out[...] = sx + st


_SC_R = 256


def _tc_score(gathered, y, m, d):
    g = B // _SC_R
    x_spec = pl.BlockSpec((_SC_R, D), lambda i: (i, 0))
    t_spec = pl.BlockSpec((_SC_R, TW), lambda i: (i, 0))
    s_spec = pl.BlockSpec((_SC_R, 1), lambda i: (i, 0))
    return pl.pallas_call(
        _tc_score_body,
        grid=(g,),
        in_specs=[x_spec] * 6 + [t_spec] * 3 + [s_spec] * 3,
        out_specs=s_spec,
        out_shape=jax.ShapeDtypeStruct((B, 1), jnp.float32),
    )(*gathered, y.reshape(B, 1), m.reshape(B, 1), d.reshape(B, 1))


def _combine(re, im):
    return jnp.concatenate(
        [jnp.concatenate([re[:, :DH], im[:, :DH]], axis=1),
         jnp.concatenate([re[:, DH:], im[:, DH:]], axis=1)], axis=0)


def kernel(heads, relations, tails, years, months, days, intervals,
           edge_index, edge_type, ent_r, ent_i, rel_r, rel_i,
           y_amp, m_amp, d_amp, y_freq, m_freq, d_freq, y_phi, m_phi, d_phi,
           W_in1, W_out1, W_loop1, w_rel1, b1, loop_r1, loop_i1,
           W_in2, W_out2, W_loop2, w_rel2, b2, loop_r2, loop_i2):
    src = edge_index[0]
    dst = edge_index[1]
    et = edge_type

    dgi, dgo = _sc_deg_kernel()(dst)

    zrows = jnp.zeros((ROW_SPAN, D), jnp.float32)
    xcat = _combine(ent_r, ent_i)
    rcat = _combine(rel_r, rel_i)
    aggi, aggo = _sc_agg_kernel()(src, dst, et, xcat, rcat, zrows)
    yr1, yi1, rrn1, rin1 = _tc_layer(
        aggi, aggo, dgi, dgo, ent_r, ent_i,
        W_in1, W_out1, W_loop1, loop_r1, loop_i1, b1, rel_r, rel_i, w_rel1)

    xcat = _combine(yr1, yi1)
    rcat = _combine(rrn1, rin1)
    aggi, aggo = _sc_agg_kernel()(src, dst, et, xcat, rcat, zrows)
    yr2, yi2, rrn2, rin2 = _tc_layer(
        aggi, aggo, dgi, dgo, yr1, yi1,
        W_in2, W_out2, W_loop2, loop_r2, loop_i2, b2, rrn1, rin1, w_rel2)

    tt = jnp.concatenate(
        [y_amp, m_amp, d_amp, y_freq, m_freq, d_freq, y_phi, m_phi, d_phi,
         jnp.zeros((NT, TW - 9 * DT), jnp.float32)], axis=1)
    gathered = _sc_gather_kernel()(
        heads, tails, relations, relations + NE, yr2, yi2, rrn2, rin2, tt)
    scores = _tc_score(gathered, years, months, days)
    return scores[:, 0]
